# baseline (speedup 1.0000x reference)
import functools

import jax
import jax.numpy as jnp
from jax import lax
from jax.experimental import pallas as pl
from jax.experimental.pallas import tpu as pltpu
from jax.experimental.pallas import tpu_sc as plsc

NC = 2
NS = 16
NW = NC * NS
L = 16

DIM = 64
BATCH = 16384
B_PER_W = BATCH // NW
GCHUNK = 128
NCHUNK = B_PER_W // GCHUNK


def _mf_body(u_idx, p_idx, n_idx, ps, ns, utab, itab,
             u_out, p_out, n_out, part_out,
             idx_v, u_rows, p_rows, n_rows, ps_v, ns_v, acc_v,
             gsem, wsem):
    wid = lax.axis_index("s") * NC + lax.axis_index("c")
    base = wid * B_PER_W

    for b, src in enumerate((u_idx, p_idx, n_idx)):
        for j in range(NCHUNK):
            pltpu.sync_copy(src.at[pl.ds(base + j * GCHUNK, GCHUNK)],
                            idx_v.at[b, j])
    pltpu.sync_copy(ps.at[pl.ds(base, B_PER_W)], ps_v)
    pltpu.sync_copy(ns.at[pl.ds(base, B_PER_W)], ns_v)

    handles = []
    for b, (tab, rows) in enumerate(((utab, u_rows), (itab, p_rows),
                                     (itab, n_rows))):
        for j in range(NCHUNK):
            handles.append(pltpu.async_copy(
                tab.at[idx_v.at[b, j]],
                rows.at[pl.ds(j * GCHUNK, GCHUNK)], gsem))
    for h in handles:
        h.wait()

    wh = [pltpu.async_copy(u_rows, u_out.at[pl.ds(base, B_PER_W)], wsem),
          pltpu.async_copy(p_rows, p_out.at[pl.ds(base, B_PER_W)], wsem),
          pltpu.async_copy(n_rows, n_out.at[pl.ds(base, B_PER_W)], wsem)]

    iota16 = lax.iota(jnp.int32, L)
    zero = jnp.zeros((L,), jnp.float32)

    def grp_body(g, carry):
        su, sp2, sn2, lpv, lnv = carry
        row_idx = g * L + iota16

        def dim_body(d, c2):
            su, sp2, sn2, dpv, dnv = c2
            col = jnp.full((L,), d, jnp.int32)
            u = plsc.load_gather(u_rows, [row_idx, col])
            p = plsc.load_gather(p_rows, [row_idx, col])
            n = plsc.load_gather(n_rows, [row_idx, col])
            return (su + u * u, sp2 + p * p, sn2 + n * n,
                    dpv + u * p, dnv + u * n)

        su, sp2, sn2, dpv, dnv = lax.fori_loop(
            0, DIM, dim_body, (su, sp2, sn2, zero, zero), unroll=8)
        ep = dpv - ps_v[pl.ds(g * L, L)]
        en = dnv - ns_v[pl.ds(g * L, L)]
        return su, sp2, sn2, lpv + ep * ep, lnv + en * en

    su, sp2, sn2, lpv, lnv = lax.fori_loop(
        0, B_PER_W // L, grp_body, (zero, zero, zero, zero, zero))

    acc_v[0, :] = su
    acc_v[1, :] = sp2
    acc_v[2, :] = sn2
    acc_v[3, :] = lpv
    acc_v[4, :] = lnv
    pltpu.sync_copy(acc_v, part_out.at[wid])

    for h in wh:
        h.wait()


@jax.jit
def _mf_call(u_idx, p_idx, n_idx, ps, ns, utab, itab):
    mesh = plsc.VectorSubcoreMesh(core_axis_name="c", subcore_axis_name="s",
                                  num_cores=NC, num_subcores=NS)
    f = pl.kernel(
        _mf_body,
        out_type=(
            jax.ShapeDtypeStruct((BATCH, DIM), jnp.float32),
            jax.ShapeDtypeStruct((BATCH, DIM), jnp.float32),
            jax.ShapeDtypeStruct((BATCH, DIM), jnp.float32),
            jax.ShapeDtypeStruct((NW, 5, L), jnp.float32),
        ),
        mesh=mesh,
        scratch_types=(
            pltpu.VMEM((3, NCHUNK, GCHUNK), jnp.int32),
            pltpu.VMEM((B_PER_W, DIM), jnp.float32),
            pltpu.VMEM((B_PER_W, DIM), jnp.float32),
            pltpu.VMEM((B_PER_W, DIM), jnp.float32),
            pltpu.VMEM((B_PER_W,), jnp.float32),
            pltpu.VMEM((B_PER_W,), jnp.float32),
            pltpu.VMEM((5, L), jnp.float32),
            pltpu.SemaphoreType.DMA,
            pltpu.SemaphoreType.DMA,
        ),
        compiler_params=pltpu.CompilerParams(needs_layout_passes=False,
                                             use_tc_tiling_on_sc=False),
        name="mf_sc_kernel",
    )
    return f(u_idx, p_idx, n_idx, ps, ns, utab, itab)


def kernel(user_list, pos_item_list, neg_item_list, pos_scores, neg_scores,
           user_table, item_table):
    u_idx = user_list.astype(jnp.int32)
    p_idx = pos_item_list.astype(jnp.int32)
    n_idx = neg_item_list.astype(jnp.int32)
    user_emb, pos_emb, neg_emb, parts = _mf_call(
        u_idx, p_idx, n_idx, pos_scores, neg_scores, user_table, item_table)
    su = jnp.sum(parts[:, 0, :])
    sp2 = jnp.sum(parts[:, 1, :])
    sn2 = jnp.sum(parts[:, 2, :])
    lp = jnp.sum(parts[:, 3, :])
    ln = jnp.sum(parts[:, 4, :])
    inv_b = jnp.float32(1.0 / BATCH)
    reg = (su + sp2 + sn2) * inv_b
    rating_loss = (lp + ln) * inv_b
    return user_emb, pos_emb, neg_emb, reg, rating_loss


# speedup vs baseline: 1.3004x; 1.3004x over previous
import jax
import jax.numpy as jnp
from jax import lax
from jax.experimental import pallas as pl
from jax.experimental.pallas import tpu as pltpu
from jax.experimental.pallas import tpu_sc as plsc

NC = 2
NS = 16
NW = NC * NS
L = 16

DIM = 64
BATCH = 16384
B_PER_W = BATCH // NW
CH = 256
NCH = B_PER_W // CH
NGRP = CH // L


def _mf_body(u_idx, p_idx, n_idx, ps, ns, utab, itab,
             u_out, p_out, n_out, part_out,
             u_idx_v, p_idx_v, n_idx_v, u_rows, p_rows, n_rows,
             ps_v, ns_v, acc_v, gsem, wsem):
    wid = lax.axis_index("s") * NC + lax.axis_index("c")
    base = wid * B_PER_W

    sh = [pltpu.async_copy(src.at[pl.ds(base, B_PER_W)], dst, gsem)
          for src, dst in ((u_idx, u_idx_v), (p_idx, p_idx_v),
                           (n_idx, n_idx_v))]
    sh.append(pltpu.async_copy(ps.at[pl.ds(base, B_PER_W)], ps_v, gsem))
    sh.append(pltpu.async_copy(ns.at[pl.ds(base, B_PER_W)], ns_v, gsem))
    for h in sh:
        h.wait()

    iota16 = lax.iota(jnp.int32, L)
    zero = jnp.zeros((L,), jnp.float32)

    def fire_grp(cbase, g):
        gb = cbase + g * L
        uvec = u_idx_v[pl.ds(gb, L)]
        pvec = p_idx_v[pl.ds(gb, L)]
        nvec = n_idx_v[pl.ds(gb, L)]
        r0 = g * L
        for i in range(L):
            pltpu.async_copy(
                utab.at[pl.ds(uvec[i], 1)],
                u_rows.at[pl.ds(r0 + i, 1)], gsem)
            pltpu.async_copy(
                itab.at[pl.ds(pvec[i], 1)],
                p_rows.at[pl.ds(r0 + i, 1)], gsem)
            pltpu.async_copy(
                itab.at[pl.ds(nvec[i], 1)],
                n_rows.at[pl.ds(r0 + i, 1)], gsem)

    def drain_grp(g):
        r0 = g * L
        for rows in (u_rows, p_rows, n_rows):
            pltpu.make_async_copy(
                utab.at[pl.ds(0, L)],
                rows.at[pl.ds(r0, L)], gsem).wait()

    carry = (zero, zero, zero, zero, zero)
    wh = []
    for j in range(NCH):
        cbase = j * CH
        for h in wh:
            h.wait()
        wh = []
        fire_grp(cbase, 0)

        def pipe_body(g, tot, cbase=cbase):
            fire_grp(cbase, g)
            drain_grp(g - 1)
            return tot

        lax.fori_loop(1, NGRP, pipe_body, 0)
        drain_grp(NGRP - 1)

        ob = base + cbase
        wh.append(pltpu.async_copy(
            u_rows, u_out.at[pl.ds(ob, CH)], wsem))
        wh.append(pltpu.async_copy(
            p_rows, p_out.at[pl.ds(ob, CH)], wsem))
        wh.append(pltpu.async_copy(
            n_rows, n_out.at[pl.ds(ob, CH)], wsem))

        def grp_body(g, carry, cbase=cbase):
            su, sp2, sn2, lpv, lnv = carry
            row_idx = g * L + iota16

            def dim_body(d, c2):
                su, sp2, sn2, dpv, dnv = c2
                col = jnp.full((L,), d, jnp.int32)
                u = plsc.load_gather(u_rows, [row_idx, col])
                p = plsc.load_gather(p_rows, [row_idx, col])
                n = plsc.load_gather(n_rows, [row_idx, col])
                return (su + u * u, sp2 + p * p, sn2 + n * n,
                        dpv + u * p, dnv + u * n)

            su, sp2, sn2, dpv, dnv = lax.fori_loop(
                0, DIM, dim_body, (su, sp2, sn2, zero, zero), unroll=8)
            ep = dpv - ps_v[pl.ds(cbase + g * L, L)]
            en = dnv - ns_v[pl.ds(cbase + g * L, L)]
            return su, sp2, sn2, lpv + ep * ep, lnv + en * en

        carry = lax.fori_loop(0, NGRP, grp_body, carry)

    su, sp2, sn2, lpv, lnv = carry
    acc_v[0, :] = su
    acc_v[1, :] = sp2
    acc_v[2, :] = sn2
    acc_v[3, :] = lpv
    acc_v[4, :] = lnv
    pltpu.sync_copy(acc_v, part_out.at[wid])

    for h in wh:
        h.wait()


@jax.jit
def _mf_call(u_idx, p_idx, n_idx, ps, ns, utab, itab):
    mesh = plsc.VectorSubcoreMesh(core_axis_name="c", subcore_axis_name="s",
                                  num_cores=NC, num_subcores=NS)
    f = pl.kernel(
        _mf_body,
        out_type=(
            jax.ShapeDtypeStruct((BATCH, DIM), jnp.float32),
            jax.ShapeDtypeStruct((BATCH, DIM), jnp.float32),
            jax.ShapeDtypeStruct((BATCH, DIM), jnp.float32),
            jax.ShapeDtypeStruct((NW, 5, L), jnp.float32),
        ),
        mesh=mesh,
        scratch_types=(
            pltpu.VMEM((B_PER_W,), jnp.int32),
            pltpu.VMEM((B_PER_W,), jnp.int32),
            pltpu.VMEM((B_PER_W,), jnp.int32),
            pltpu.VMEM((CH, DIM), jnp.float32),
            pltpu.VMEM((CH, DIM), jnp.float32),
            pltpu.VMEM((CH, DIM), jnp.float32),
            pltpu.VMEM((B_PER_W,), jnp.float32),
            pltpu.VMEM((B_PER_W,), jnp.float32),
            pltpu.VMEM((5, L), jnp.float32),
            pltpu.SemaphoreType.DMA,
            pltpu.SemaphoreType.DMA,
        ),
        compiler_params=pltpu.CompilerParams(needs_layout_passes=False,
                                             use_tc_tiling_on_sc=True),
        name="mf_sc_kernel",
    )
    return f(u_idx, p_idx, n_idx, ps, ns, utab, itab)


def kernel(user_list, pos_item_list, neg_item_list, pos_scores, neg_scores,
           user_table, item_table):
    u_idx = user_list.astype(jnp.int32)
    p_idx = pos_item_list.astype(jnp.int32)
    n_idx = neg_item_list.astype(jnp.int32)
    user_emb, pos_emb, neg_emb, parts = _mf_call(
        u_idx, p_idx, n_idx, pos_scores, neg_scores, user_table, item_table)
    su = jnp.sum(parts[:, 0, :])
    sp2 = jnp.sum(parts[:, 1, :])
    sn2 = jnp.sum(parts[:, 2, :])
    lp = jnp.sum(parts[:, 3, :])
    ln = jnp.sum(parts[:, 4, :])
    inv_b = jnp.float32(1.0 / BATCH)
    reg = (su + sp2 + sn2) * inv_b
    rating_loss = (lp + ln) * inv_b
    return user_emb, pos_emb, neg_emb, reg, rating_loss


# speedup vs baseline: 2.8003x; 2.1534x over previous
import jax
import jax.numpy as jnp
from jax import lax
from jax.experimental import pallas as pl
from jax.experimental.pallas import tpu as pltpu
from jax.experimental.pallas import tpu_sc as plsc

NC = 2
NS = 16
NW = NC * NS
L = 16

DIM = 64
BATCH = 16384
B_PER_W = BATCH // NW
CH = 128
NCH = B_PER_W // CH
NGRP = CH // L


def _mf_body(u_idx, p_idx, n_idx, ps, ns, utab, itab,
             u_out, p_out, n_out, part_out,
             u_idx_v, p_idx_v, n_idx_v, rows0, rows1,
             ps_v, ns_v, acc_v, gsem, wsem):
    wid = lax.axis_index("s") * NC + lax.axis_index("c")
    base = wid * B_PER_W

    sh = [pltpu.async_copy(src.at[pl.ds(base, B_PER_W)], dst, gsem)
          for src, dst in ((u_idx, u_idx_v), (p_idx, p_idx_v),
                           (n_idx, n_idx_v))]
    sh.append(pltpu.async_copy(ps.at[pl.ds(base, B_PER_W)], ps_v, gsem))
    sh.append(pltpu.async_copy(ns.at[pl.ds(base, B_PER_W)], ns_v, gsem))
    for h in sh:
        h.wait()

    iota16 = lax.iota(jnp.int32, L)
    zero = jnp.zeros((L,), jnp.float32)
    bufs = (rows0, rows1)

    def fire_grp(buf, cbase, g):
        gb = cbase + g * L
        uvec = u_idx_v[pl.ds(gb, L)]
        pvec = p_idx_v[pl.ds(gb, L)]
        nvec = n_idx_v[pl.ds(gb, L)]
        r0 = g * L
        for i in range(L):
            pltpu.async_copy(utab.at[pl.ds(uvec[i], 1)],
                             buf.at[0, pl.ds(r0 + i, 1)], gsem)
            pltpu.async_copy(itab.at[pl.ds(pvec[i], 1)],
                             buf.at[1, pl.ds(r0 + i, 1)], gsem)
            pltpu.async_copy(itab.at[pl.ds(nvec[i], 1)],
                             buf.at[2, pl.ds(r0 + i, 1)], gsem)

    def fire_chunk(buf, cbase):
        def pipe_body(g, tot):
            fire_grp(buf, cbase, g)
            return tot
        fire_grp(buf, cbase, 0)
        lax.fori_loop(1, NGRP, pipe_body, 0)

    def drain_chunk(buf):
        for b in range(3):
            pltpu.make_async_copy(
                utab.at[pl.ds(0, CH)], buf.at[b], gsem).wait()

    def compute_chunk(buf, cbase, carry):
        def grp_body(g, carry):
            su, sp2, sn2, lpv, lnv = carry
            row_idx = g * L + iota16

            def dim_body(d, c2):
                su, sp2, sn2, dpv, dnv = c2
                col = jnp.full((L,), d, jnp.int32)
                u = plsc.load_gather(buf.at[0], [row_idx, col])
                p = plsc.load_gather(buf.at[1], [row_idx, col])
                n = plsc.load_gather(buf.at[2], [row_idx, col])
                return (su + u * u, sp2 + p * p, sn2 + n * n,
                        dpv + u * p, dnv + u * n)

            su, sp2, sn2, dpv, dnv = lax.fori_loop(
                0, DIM, dim_body, (su, sp2, sn2, zero, zero), unroll=8)
            ep = dpv - ps_v[pl.ds(cbase + g * L, L)]
            en = dnv - ns_v[pl.ds(cbase + g * L, L)]
            return su, sp2, sn2, lpv + ep * ep, lnv + en * en

        return lax.fori_loop(0, NGRP, grp_body, carry)

    def write_chunk(buf, cbase):
        ob = base + cbase
        return [
            pltpu.async_copy(buf.at[0], u_out.at[pl.ds(ob, CH)], wsem),
            pltpu.async_copy(buf.at[1], p_out.at[pl.ds(ob, CH)], wsem),
            pltpu.async_copy(buf.at[2], n_out.at[pl.ds(ob, CH)], wsem),
        ]

    carry = (zero, zero, zero, zero, zero)
    fire_chunk(bufs[0], 0)
    wh = []
    for j in range(NCH):
        buf = bufs[j % 2]
        drain_chunk(buf)
        if j + 1 < NCH:
            nbuf = bufs[(j + 1) % 2]
            for h in wh:
                h.wait()
            fire_chunk(nbuf, (j + 1) * CH)
        whn = write_chunk(buf, j * CH)
        carry = compute_chunk(buf, j * CH, carry)
        wh = whn

    su, sp2, sn2, lpv, lnv = carry
    acc_v[0, :] = su
    acc_v[1, :] = sp2
    acc_v[2, :] = sn2
    acc_v[3, :] = lpv
    acc_v[4, :] = lnv
    pltpu.sync_copy(acc_v, part_out.at[wid])

    for h in wh:
        h.wait()


@jax.jit
def _mf_call(u_idx, p_idx, n_idx, ps, ns, utab, itab):
    mesh = plsc.VectorSubcoreMesh(core_axis_name="c", subcore_axis_name="s",
                                  num_cores=NC, num_subcores=NS)
    f = pl.kernel(
        _mf_body,
        out_type=(
            jax.ShapeDtypeStruct((BATCH, DIM), jnp.float32),
            jax.ShapeDtypeStruct((BATCH, DIM), jnp.float32),
            jax.ShapeDtypeStruct((BATCH, DIM), jnp.float32),
            jax.ShapeDtypeStruct((NW, 5, L), jnp.float32),
        ),
        mesh=mesh,
        scratch_types=(
            pltpu.VMEM((B_PER_W,), jnp.int32),
            pltpu.VMEM((B_PER_W,), jnp.int32),
            pltpu.VMEM((B_PER_W,), jnp.int32),
            pltpu.VMEM((3, CH, DIM), jnp.float32),
            pltpu.VMEM((3, CH, DIM), jnp.float32),
            pltpu.VMEM((B_PER_W,), jnp.float32),
            pltpu.VMEM((B_PER_W,), jnp.float32),
            pltpu.VMEM((5, L), jnp.float32),
            pltpu.SemaphoreType.DMA,
            pltpu.SemaphoreType.DMA,
        ),
        compiler_params=pltpu.CompilerParams(needs_layout_passes=False,
                                             use_tc_tiling_on_sc=True),
        name="mf_sc_kernel",
    )
    return f(u_idx, p_idx, n_idx, ps, ns, utab, itab)


def kernel(user_list, pos_item_list, neg_item_list, pos_scores, neg_scores,
           user_table, item_table):
    u_idx = user_list.astype(jnp.int32)
    p_idx = pos_item_list.astype(jnp.int32)
    n_idx = neg_item_list.astype(jnp.int32)
    user_emb, pos_emb, neg_emb, parts = _mf_call(
        u_idx, p_idx, n_idx, pos_scores, neg_scores, user_table, item_table)
    su = jnp.sum(parts[:, 0, :])
    sp2 = jnp.sum(parts[:, 1, :])
    sn2 = jnp.sum(parts[:, 2, :])
    lp = jnp.sum(parts[:, 3, :])
    ln = jnp.sum(parts[:, 4, :])
    inv_b = jnp.float32(1.0 / BATCH)
    reg = (su + sp2 + sn2) * inv_b
    rating_loss = (lp + ln) * inv_b
    return user_emb, pos_emb, neg_emb, reg, rating_loss
